# baseline (speedup 1.0000x reference)
import functools

import jax
import jax.numpy as jnp
from jax import lax
from jax.experimental import pallas as pl
from jax.experimental.pallas import tpu as pltpu
from jax.experimental.pallas import tpu_sc as plsc

P, N, W = 26, 16384, 50
NUM_WORKERS = 32
NCOL = 16
COLW = N // NCOL
PHALF = P // 2
FULL_SLABS = W // 8
TAIL_ROWS = W - 8 * FULL_SLABS
LANES = 16
UNROLL = 4


def _sc_fill(ctx_t, flag_t, ids_t, table):
    mesh = plsc.VectorSubcoreMesh(core_axis_name="c", subcore_axis_name="s")

    @functools.partial(
        pl.kernel,
        mesh=mesh,
        out_type=jax.ShapeDtypeStruct((P, W, N), jnp.float32),
        scratch_types=[
            pltpu.VMEM((8 * COLW,), jnp.int32),
            pltpu.VMEM((8 * COLW,), jnp.float32),
            pltpu.VMEM((8, COLW), jnp.int32),
            pltpu.VMEM((8, COLW), jnp.float32),
            pltpu.VMEM((TAIL_ROWS * COLW,), jnp.int32),
            pltpu.VMEM((TAIL_ROWS * COLW,), jnp.float32),
            pltpu.VMEM((TAIL_ROWS, COLW), jnp.int32),
            pltpu.VMEM((TAIL_ROWS, COLW), jnp.float32),
        ]
        + [pltpu.SemaphoreType.DMA] * 2,
    )
    def k(ctx_hbm, flag_hbm, ids_hbm, tab_hbm, out_hbm,
          idx_v, g_v, f_v, c_v, idx_t, g_t, f_t, c_t, sg, so):
        wid = lax.axis_index("s") * 2 + lax.axis_index("c")
        col = wid % NCOL
        half = wid // NCOL
        n0 = col * COLW
        p0 = half * PHALF

        def do_slab(p, r0, nrows, idx_b, g_b, f_b, c_b):
            rows = pl.ds(r0, nrows)
            cols = pl.ds(n0, COLW)
            for r in range(nrows):
                pltpu.sync_copy(ids_hbm.at[p, r0 + r, cols],
                                idx_b.at[pl.ds(r * COLW, COLW)])
            pltpu.async_copy(tab_hbm.at[idx_b], g_b, sg).wait()
            pltpu.sync_copy(flag_hbm.at[p, rows, cols], f_b)
            pltpu.sync_copy(ctx_hbm.at[p, rows, cols], c_b)

            for r in range(nrows):
                def vec_body(j, carry):
                    base = j * (LANES * UNROLL)
                    for u in range(UNROLL):
                        s = pl.ds(base + u * LANES, LANES)
                        gs = pl.ds(r * COLW + base + u * LANES, LANES)
                        c_b[r, s] = jnp.where(
                            f_b[r, s] == 1, g_v_read(g_b, gs), c_b[r, s])
                    return carry

                lax.fori_loop(0, COLW // (LANES * UNROLL), vec_body, 0)

            pltpu.async_copy(c_b, out_hbm.at[p, rows, cols], so).wait()

        def g_v_read(g_b, gs):
            return g_b[gs]

        def p_body(i, carry):
            p = p0 + i
            for t in range(FULL_SLABS):
                do_slab(p, 8 * t, 8, idx_v, g_v, f_v, c_v)
            do_slab(p, 8 * FULL_SLABS, TAIL_ROWS, idx_t, g_t, f_t, c_t)
            return carry

        lax.fori_loop(0, PHALF, p_body, 0)

    return k(ctx_t, flag_t, ids_t, table)


def kernel(contexts, missing_flag, cell_ids, learning_cell):
    def tview(x):
        return jnp.transpose(x, (0, 2, 1))

    filled_t = _sc_fill(
        tview(contexts), tview(missing_flag),
        tview(cell_ids.astype(jnp.int32)), learning_cell,
    )
    return jnp.transpose(filled_t, (0, 2, 1)), learning_cell


# speedup vs baseline: 1.7625x; 1.7625x over previous
import functools

import jax
import jax.numpy as jnp
from jax import lax
from jax.experimental import pallas as pl
from jax.experimental.pallas import tpu as pltpu
from jax.experimental.pallas import tpu_sc as plsc

P, N, W = 26, 16384, 50
NUM_WORKERS = 32
NCOL = 16
COLW = N // NCOL
PHALF = P // 2
FULL_SLABS = W // 8
TAIL_ROWS = W - 8 * FULL_SLABS
NSTEP = PHALF * FULL_SLABS
SLAB = 8 * COLW
TAIL = TAIL_ROWS * COLW
LANES = 16
UNROLL = 4


def _sc_fill(ctx_t, flag_t, ids_t, table):
    mesh = plsc.VectorSubcoreMesh(core_axis_name="c", subcore_axis_name="s")

    @functools.partial(
        pl.kernel,
        mesh=mesh,
        out_type=jax.ShapeDtypeStruct((P, W, N), jnp.float32),
        scratch_types=[pltpu.VMEM((SLAB,), jnp.int32)] * 3
        + [pltpu.VMEM((SLAB,), jnp.float32)] * 2
        + [pltpu.VMEM((8, COLW), jnp.int32)] * 2
        + [pltpu.VMEM((8, COLW), jnp.float32)] * 2
        + [
            pltpu.VMEM((TAIL,), jnp.int32),
            pltpu.VMEM((TAIL,), jnp.float32),
            pltpu.VMEM((TAIL_ROWS, COLW), jnp.int32),
            pltpu.VMEM((TAIL_ROWS, COLW), jnp.float32),
        ]
        + [pltpu.SemaphoreType.DMA] * 13,
    )
    def k(ctx_hbm, flag_hbm, ids_hbm, tab_hbm, out_hbm,
          i0, i1, i2, g0, g1, f0, f1, c0, c1, idx_t, g_t, f_t, c_t,
          si0, si1, si2, sg0, sg1, sf0, sf1, sc0, sc1, so0, so1, stg, sto):
        idx_v = (i0, i1, i2)
        g_v = (g0, g1)
        f_v = (f0, f1)
        c_v = (c0, c1)
        s_idx = (si0, si1, si2)
        s_g = (sg0, sg1)
        s_f = (sf0, sf1)
        s_c = (sc0, sc1)
        s_o = (so0, so1)
        wid = lax.axis_index("s") * 2 + lax.axis_index("c")
        col = wid % NCOL
        half = wid // NCOL
        n0 = col * COLW
        p0 = half * PHALF
        cols = pl.ds(n0, COLW)

        def slab_pr(i):
            return p0 + i // FULL_SLABS, 8 * (i % FULL_SLABS)

        def start_ids(i, s3):
            p, r0 = slab_pr(i)
            for r in range(8):
                pltpu.async_copy(ids_hbm.at[p, r0 + r, cols],
                                 idx_v[s3].at[pl.ds(r * COLW, COLW)],
                                 s_idx[s3])

        def wait_ids(s3):
            pltpu.make_async_copy(
                tab_hbm.at[pl.ds(0, SLAB)], idx_v[s3], s_idx[s3]).wait()

        def start_gather(s3, s2):
            pltpu.async_copy(tab_hbm.at[idx_v[s3]], g_v[s2], s_g[s2])

        def wait_gather(s3, s2):
            pltpu.make_async_copy(tab_hbm.at[idx_v[s3]], g_v[s2],
                                  s_g[s2]).wait()

        def start_fc(i, s2):
            p, r0 = slab_pr(i)
            rows = pl.ds(r0, 8)
            pltpu.async_copy(flag_hbm.at[p, rows, cols], f_v[s2], s_f[s2])
            pltpu.async_copy(ctx_hbm.at[p, rows, cols], c_v[s2], s_c[s2])

        def wait_fc(i, s2):
            p, r0 = slab_pr(i)
            rows = pl.ds(r0, 8)
            pltpu.make_async_copy(
                flag_hbm.at[p, rows, cols], f_v[s2], s_f[s2]).wait()
            pltpu.make_async_copy(
                ctx_hbm.at[p, rows, cols], c_v[s2], s_c[s2]).wait()

        def start_out(i, s2):
            p, r0 = slab_pr(i)
            pltpu.async_copy(c_v[s2], out_hbm.at[p, pl.ds(r0, 8), cols],
                             s_o[s2])

        def wait_out(i, s2):
            p, r0 = slab_pr(i)
            pltpu.make_async_copy(
                c_v[s2], out_hbm.at[p, pl.ds(r0, 8), cols], s_o[s2]).wait()

        def select(s2):
            fb, gb, cb = f_v[s2], g_v[s2], c_v[s2]

            def vec_body(j, carry):
                base = j * (LANES * UNROLL)
                for r in range(8):
                    for u in range(UNROLL):
                        o = base + u * LANES
                        s = pl.ds(o, LANES)
                        cb[r, s] = jnp.where(
                            fb[r, s] == 1,
                            gb[pl.ds(r * COLW + o, LANES)], cb[r, s])
                return carry

            lax.fori_loop(0, COLW // (LANES * UNROLL), vec_body, 0)

        def step(i, b3, b2, w_out_prev, do_next, do_pf2):
            if w_out_prev:
                wait_out(i - 1, (b2 - 1) % 2)
            if do_next:
                wait_ids((b3 + 1) % 3)
                start_gather((b3 + 1) % 3, (b2 + 1) % 2)
                start_fc(i + 1, (b2 + 1) % 2)
            if do_pf2:
                start_ids(i + 2, (b3 + 2) % 3)
            wait_gather(b3, b2)
            wait_fc(i, b2)
            select(b2)
            start_out(i, b2)

        start_ids(0, 0)
        start_ids(1, 1)
        wait_ids(0)
        start_gather(0, 0)
        start_fc(0, 0)

        step(0, 0, 0, False, True, True)
        for kk in range(1, 6):
            step(kk, kk % 3, kk % 2, True, True, True)

        def group_body(g, carry):
            i6 = g * 6
            for kk in range(6):
                step(i6 + kk, kk % 3, kk % 2, True, True, True)
            return carry

        lax.fori_loop(1, NSTEP // 6 - 1, group_body, 0)

        iL = NSTEP - 6
        for kk in range(4):
            step(iL + kk, kk % 3, kk % 2, True, True, True)
        step(iL + 4, 4 % 3, 4 % 2, True, True, False)
        step(iL + 5, 5 % 3, 5 % 2, True, False, False)
        wait_out(NSTEP - 1, (NSTEP - 1) % 2)

        def tail_body(ip, carry):
            p = p0 + ip
            rows = pl.ds(8 * FULL_SLABS, TAIL_ROWS)
            for r in range(TAIL_ROWS):
                pltpu.sync_copy(ids_hbm.at[p, 8 * FULL_SLABS + r, cols],
                                idx_t.at[pl.ds(r * COLW, COLW)])
            pltpu.async_copy(tab_hbm.at[idx_t], g_t, stg).wait()
            pltpu.sync_copy(flag_hbm.at[p, rows, cols], f_t)
            pltpu.sync_copy(ctx_hbm.at[p, rows, cols], c_t)

            def vec_body(j, carry2):
                base = j * (LANES * UNROLL)
                for r in range(TAIL_ROWS):
                    for u in range(UNROLL):
                        o = base + u * LANES
                        s = pl.ds(o, LANES)
                        c_t[r, s] = jnp.where(
                            f_t[r, s] == 1,
                            g_t[pl.ds(r * COLW + o, LANES)], c_t[r, s])
                return carry2

            lax.fori_loop(0, COLW // (LANES * UNROLL), vec_body, 0)
            pltpu.async_copy(c_t, out_hbm.at[p, rows, cols], sto).wait()
            return carry

        lax.fori_loop(0, PHALF, tail_body, 0)

    return k(ctx_t, flag_t, ids_t, table)


def kernel(contexts, missing_flag, cell_ids, learning_cell):
    def tview(x):
        return jnp.transpose(x, (0, 2, 1))

    filled_t = _sc_fill(
        tview(contexts), tview(missing_flag),
        tview(cell_ids.astype(jnp.int32)), learning_cell,
    )
    return jnp.transpose(filled_t, (0, 2, 1)), learning_cell
